# baseline (speedup 1.0000x reference)
import functools

import jax
import jax.numpy as jnp
from jax import lax
from jax.experimental import pallas as pl
from jax.experimental.pallas import tpu as pltpu
from jax.experimental.pallas import tpu_sc as plsc

N_NODES = 100000
ATOM_DIM = 32
CHARGE_DIM = 16
OUT_DIM = ATOM_DIM + CHARGE_DIM
SUPER = 128 // ATOM_DIM
A_SUPER_ROWS = 1000000 // SUPER
C_FLAT = 100 * CHARGE_DIM

NUM_CORES = 2
NUM_SUBCORES = 16
NW = NUM_CORES * NUM_SUBCORES
L = 16

BPW = 3200
LAST_BASE = N_NODES - BPW
CHUNK = 320
NCHUNK = BPW // CHUNK

_mesh = plsc.VectorSubcoreMesh(core_axis_name="c", subcore_axis_name="s")


@functools.partial(
    pl.kernel,
    mesh=_mesh,
    compiler_params=pltpu.CompilerParams(needs_layout_passes=False),
    out_type=jax.ShapeDtypeStruct((N_NODES, OUT_DIM), jnp.float32),
    scratch_types=[
        pltpu.VMEM((BPW,), jnp.int32),
        pltpu.VMEM((BPW,), jnp.int32),
        pltpu.VMEM((BPW,), jnp.int32),
        pltpu.VMEM((C_FLAT,), jnp.float32),
        pltpu.VMEM((CHUNK, 128), jnp.float32),
        pltpu.VMEM((CHUNK, OUT_DIM), jnp.float32),
        pltpu.SemaphoreType.DMA,
    ],
)
def _sc_embed(aidx_hbm, cidx_hbm, atable_hbm, ctable_hbm, out_hbm,
              asup_v, aoff_v, cidx_v, ctab_v, apad_v, comb_v, sem):
    wid = lax.axis_index("s") * NUM_CORES + lax.axis_index("c")
    base = jnp.minimum(wid * BPW, LAST_BASE)
    pltpu.sync_copy(aidx_hbm.at[pl.ds(base, BPW)], asup_v)
    pltpu.sync_copy(cidx_hbm.at[pl.ds(base, BPW)], cidx_v)
    pltpu.sync_copy(ctable_hbm, ctab_v)

    def prep(g, carry):
        v = asup_v[pl.ds(g * L, L)]
        aoff_v[pl.ds(g * L, L)] = (v & 3) << 5
        asup_v[pl.ds(g * L, L)] = jax.lax.shift_right_logical(v, 2)
        return carry

    lax.fori_loop(0, BPW // L, prep, 0)

    lanes = lax.iota(jnp.int32, L)

    def body(j, carry):
        off = j * CHUNK
        pltpu.async_copy(
            atable_hbm.at[asup_v.at[pl.ds(off, CHUNK)]], apad_v, sem).wait()

        def group(g, c2):
            rows = g * L + lanes
            aoff = aoff_v[pl.ds(off + g * L, L)]
            cflat = cidx_v[pl.ds(off + g * L, L)] * CHARGE_DIM
            for col in range(ATOM_DIM):
                v = plsc.load_gather(apad_v, [rows, aoff + col])
                plsc.store_scatter(
                    comb_v, [rows, jnp.full((L,), col, jnp.int32)], v)
            for col in range(CHARGE_DIM):
                v = plsc.load_gather(ctab_v, [cflat + col])
                plsc.store_scatter(
                    comb_v,
                    [rows, jnp.full((L,), ATOM_DIM + col, jnp.int32)], v)
            return c2

        lax.fori_loop(0, CHUNK // L, group, 0)
        pltpu.sync_copy(comb_v, out_hbm.at[pl.ds(base + off, CHUNK)])
        return carry

    lax.fori_loop(0, NCHUNK, body, 0)


def kernel(atom_types, charge, atom_types_table, charge_table):
    aidx = atom_types.reshape(-1).astype(jnp.int32)
    cidx = charge.reshape(-1).astype(jnp.int32)
    atable2 = atom_types_table.reshape(A_SUPER_ROWS, 128)
    ctable1 = charge_table.reshape(C_FLAT)
    return _sc_embed(aidx, cidx, atable2, ctable1)


# speedup vs baseline: 1.1959x; 1.1959x over previous
import functools

import jax
import jax.numpy as jnp
from jax import lax
from jax.experimental import pallas as pl
from jax.experimental.pallas import tpu as pltpu
from jax.experimental.pallas import tpu_sc as plsc

N_NODES = 100000
ATOM_DIM = 32
CHARGE_DIM = 16
OUT_DIM = ATOM_DIM + CHARGE_DIM
LANE = 128

NUM_CORES = 2
NUM_SUBCORES = 16
NW = NUM_CORES * NUM_SUBCORES

BPW = 3200
LAST_BASE = N_NODES - BPW
CHUNK = 400
NCHUNK = BPW // CHUNK

_mesh = plsc.VectorSubcoreMesh(core_axis_name="c", subcore_axis_name="s")


@functools.partial(
    pl.kernel,
    mesh=_mesh,
    compiler_params=pltpu.CompilerParams(use_tc_tiling_on_sc=False),
    out_type=jax.ShapeDtypeStruct((N_NODES // 8, 8, LANE), jnp.float32),
    scratch_types=[
        pltpu.VMEM((BPW,), jnp.int32),
        pltpu.VMEM((BPW,), jnp.int32),
        pltpu.VMEM((CHUNK, ATOM_DIM), jnp.float32),
        pltpu.VMEM((CHUNK, CHARGE_DIM), jnp.float32),
        pltpu.VMEM((CHUNK // 8, 8, LANE), jnp.float32),
        pltpu.SemaphoreType.DMA,
        pltpu.SemaphoreType.DMA,
    ],
)
def _sc_embed(aidx_hbm, cidx_hbm, atable_hbm, ctable_hbm, out_hbm,
              aidx_v, cidx_v, arows_v, crows_v, comb_v, sem_a, sem_c):
    wid = lax.axis_index("s") * NUM_CORES + lax.axis_index("c")
    base = jnp.minimum(wid * BPW, LAST_BASE)
    pltpu.sync_copy(aidx_hbm.at[pl.ds(base, BPW)], aidx_v)
    pltpu.sync_copy(cidx_hbm.at[pl.ds(base, BPW)], cidx_v)

    def body(j, carry):
        off = j * CHUNK
        cp_a = pltpu.async_copy(
            atable_hbm.at[aidx_v.at[pl.ds(off, CHUNK)]], arows_v, sem_a)
        cp_c = pltpu.async_copy(
            ctable_hbm.at[cidx_v.at[pl.ds(off, CHUNK)]], crows_v, sem_c)
        cp_a.wait()
        cp_c.wait()

        def merge(g, c2):
            for s in range(8):
                r = g * 8 + s
                comb_v[g, s, pl.ds(0, 16)] = arows_v[r, pl.ds(0, 16)]
                comb_v[g, s, pl.ds(16, 16)] = arows_v[r, pl.ds(16, 16)]
                comb_v[g, s, pl.ds(32, 16)] = crows_v[r, pl.ds(0, 16)]
            return c2

        lax.fori_loop(0, CHUNK // 8, merge, 0)
        pltpu.sync_copy(comb_v, out_hbm.at[pl.ds((base + off) // 8, CHUNK // 8)])
        return carry

    lax.fori_loop(0, NCHUNK, body, 0)


def kernel(atom_types, charge, atom_types_table, charge_table):
    aidx = atom_types.reshape(-1).astype(jnp.int32)
    cidx = charge.reshape(-1).astype(jnp.int32)
    out3 = _sc_embed(aidx, cidx, atom_types_table, charge_table)
    return out3.reshape(N_NODES, LANE)[:, :OUT_DIM]
